# baseline (speedup 1.0000x reference)
import functools

import jax
import jax.numpy as jnp
from jax import lax
from jax.experimental import pallas as pl
from jax.experimental.pallas import tpu as pltpu
from jax.experimental.pallas import tpu_sc as plsc

B = 16384
D_IN = 256
OBJ_EMBED_DIM = 64
FC_DIM = 256
NUM_ATTR = 400

NC = 2
NS = 16
NW = NC * NS
B_PER_W = B // NW
IDX_CHUNK = 128
NCHUNK = B_PER_W // IDX_CHUNK

@functools.cache
def _get_sc_gather():
    mesh = plsc.VectorSubcoreMesh(core_axis_name="c", subcore_axis_name="s")

    @functools.partial(
        pl.kernel,
        mesh=mesh,
        out_type=jax.ShapeDtypeStruct((B, OBJ_EMBED_DIM), jnp.float32),
        scratch_types=[
            pltpu.VMEM((NCHUNK, IDX_CHUNK), jnp.int32),
            pltpu.VMEM((B_PER_W, OBJ_EMBED_DIM), jnp.float32),
            pltpu.SemaphoreType.DMA,
        ],
        compiler_params=pltpu.CompilerParams(use_tc_tiling_on_sc=False),
    )
    def _sc_gather(emb_hbm, idx_hbm, out_hbm, idx_v, rows_v, sem):
        wid = lax.axis_index("s") * NC + lax.axis_index("c")
        pltpu.sync_copy(idx_hbm.at[wid], idx_v)
        copies = [
            pltpu.async_copy(
                emb_hbm.at[idx_v.at[j]],
                rows_v.at[pl.ds(j * IDX_CHUNK, IDX_CHUNK)],
                sem,
            )
            for j in range(NCHUNK)
        ]
        for cp in copies:
            cp.wait()
        pltpu.sync_copy(rows_v, out_hbm.at[pl.ds(wid * B_PER_W, B_PER_W)])

    return _sc_gather


BLK = 512


def _mlp_body(x_ref, e_ref, wfc_ref, bfc_ref, wattr_ref, battr_ref, out_ref):
    h = jnp.dot(x_ref[:], wfc_ref[:D_IN, :], preferred_element_type=jnp.float32)
    h = h + jnp.dot(e_ref[:], wfc_ref[D_IN:, :], preferred_element_type=jnp.float32)
    h = jnp.maximum(h + bfc_ref[:], 0.0)
    out_ref[:] = (
        jnp.dot(h, wattr_ref[:], preferred_element_type=jnp.float32) + battr_ref[:]
    )


def _tc_mlp(x, e, W_fc, b_fc, W_attr, b_attr):
    return pl.pallas_call(
        _mlp_body,
        grid=(B // BLK,),
        in_specs=[
            pl.BlockSpec((BLK, D_IN), lambda i: (i, 0)),
            pl.BlockSpec((BLK, OBJ_EMBED_DIM), lambda i: (i, 0)),
            pl.BlockSpec((D_IN + OBJ_EMBED_DIM, FC_DIM), lambda i: (0, 0)),
            pl.BlockSpec((1, FC_DIM), lambda i: (0, 0)),
            pl.BlockSpec((FC_DIM, NUM_ATTR), lambda i: (0, 0)),
            pl.BlockSpec((1, NUM_ATTR), lambda i: (0, 0)),
        ],
        out_specs=pl.BlockSpec((BLK, NUM_ATTR), lambda i: (i, 0)),
        out_shape=jax.ShapeDtypeStruct((B, NUM_ATTR), jnp.float32),
    )(x, e, W_fc, b_fc, W_attr, b_attr)


def kernel(x, obj_labels, emb, W_fc, b_fc, W_attr, b_attr):
    idx = obj_labels.reshape(NW, NCHUNK, IDX_CHUNK)
    e = _get_sc_gather()(emb, idx)
    return _tc_mlp(
        x,
        e,
        W_fc,
        b_fc.reshape(1, FC_DIM),
        W_attr,
        b_attr.reshape(1, NUM_ATTR),
    )


# speedup vs baseline: 1.1044x; 1.1044x over previous
import functools

import jax
import jax.numpy as jnp
from jax import lax
from jax.experimental import pallas as pl
from jax.experimental.pallas import tpu as pltpu
from jax.experimental.pallas import tpu_sc as plsc

B = 16384
D_IN = 256
OBJ_EMBED_DIM = 64
FC_DIM = 256
NUM_ATTR = 400

NC = 2
NS = 16
NW = NC * NS
B_PER_W = B // NW
IDX_CHUNK = 128
NCHUNK = B_PER_W // IDX_CHUNK

@functools.cache
def _get_sc_gather():
    mesh = plsc.VectorSubcoreMesh(core_axis_name="c", subcore_axis_name="s")

    @functools.partial(
        pl.kernel,
        mesh=mesh,
        out_type=jax.ShapeDtypeStruct((B, OBJ_EMBED_DIM), jnp.float32),
        scratch_types=[
            pltpu.VMEM((NCHUNK, IDX_CHUNK), jnp.int32),
            pltpu.VMEM((B_PER_W, OBJ_EMBED_DIM), jnp.float32),
            pltpu.SemaphoreType.DMA,
        ],
        compiler_params=pltpu.CompilerParams(use_tc_tiling_on_sc=False),
    )
    def _sc_gather(emb_hbm, idx_hbm, out_hbm, idx_v, rows_v, sem):
        wid = lax.axis_index("s") * NC + lax.axis_index("c")
        pltpu.sync_copy(idx_hbm.at[wid], idx_v)
        copies = [
            pltpu.async_copy(
                emb_hbm.at[idx_v.at[j]],
                rows_v.at[pl.ds(j * IDX_CHUNK, IDX_CHUNK)],
                sem,
            )
            for j in range(NCHUNK)
        ]
        for cp in copies:
            cp.wait()
        pltpu.sync_copy(rows_v, out_hbm.at[pl.ds(wid * B_PER_W, B_PER_W)])

    return _sc_gather


BLK = 4096


def _mlp_body(x_ref, e_ref, wfc_ref, bfc_ref, wattr_ref, battr_ref, out_ref):
    h = jnp.dot(x_ref[:], wfc_ref[:D_IN, :], preferred_element_type=jnp.float32)
    h = h + jnp.dot(e_ref[:], wfc_ref[D_IN:, :], preferred_element_type=jnp.float32)
    h = jnp.maximum(h + bfc_ref[:], 0.0)
    out_ref[:] = (
        jnp.dot(h, wattr_ref[:], preferred_element_type=jnp.float32) + battr_ref[:]
    )


def _tc_mlp(x, e, W_fc, b_fc, W_attr, b_attr):
    return pl.pallas_call(
        _mlp_body,
        grid=(B // BLK,),
        in_specs=[
            pl.BlockSpec((BLK, D_IN), lambda i: (i, 0)),
            pl.BlockSpec((BLK, OBJ_EMBED_DIM), lambda i: (i, 0)),
            pl.BlockSpec((D_IN + OBJ_EMBED_DIM, FC_DIM), lambda i: (0, 0)),
            pl.BlockSpec((1, FC_DIM), lambda i: (0, 0)),
            pl.BlockSpec((FC_DIM, NUM_ATTR), lambda i: (0, 0)),
            pl.BlockSpec((1, NUM_ATTR), lambda i: (0, 0)),
        ],
        out_specs=pl.BlockSpec((BLK, NUM_ATTR), lambda i: (i, 0)),
        out_shape=jax.ShapeDtypeStruct((B, NUM_ATTR), jnp.float32),
    )(x, e, W_fc, b_fc, W_attr, b_attr)


def kernel(x, obj_labels, emb, W_fc, b_fc, W_attr, b_attr):
    idx = obj_labels.reshape(NW, NCHUNK, IDX_CHUNK)
    e = _get_sc_gather()(emb, idx)
    return _tc_mlp(
        x,
        e,
        W_fc,
        b_fc.reshape(1, FC_DIM),
        W_attr,
        b_attr.reshape(1, NUM_ATTR),
    )


# speedup vs baseline: 1.2153x; 1.1004x over previous
import functools

import jax
import jax.numpy as jnp
from jax import lax
from jax.experimental import pallas as pl
from jax.experimental.pallas import tpu as pltpu
from jax.experimental.pallas import tpu_sc as plsc

B = 16384
D_IN = 256
OBJ_EMBED_DIM = 64
FC_DIM = 256
NUM_ATTR = 400

NC = 2
NS = 16
NW = NC * NS
B_PER_W = B // NW
KFIRE = 16


@functools.cache
def _get_sc_gather():
    mesh = plsc.VectorSubcoreMesh(core_axis_name="c", subcore_axis_name="s")

    @functools.partial(
        pl.kernel,
        mesh=mesh,
        out_type=jax.ShapeDtypeStruct((B, OBJ_EMBED_DIM), jnp.float32),
        scratch_types=[
            pltpu.VMEM((B_PER_W,), jnp.int32),
            pltpu.VMEM((B_PER_W, OBJ_EMBED_DIM), jnp.float32),
            pltpu.SemaphoreType.DMA,
        ],
    )
    def _sc_gather(emb_hbm, idx_hbm, out_hbm, idx_v, rows_v, sem):
        wid = lax.axis_index("s") * NC + lax.axis_index("c")
        base = wid * B_PER_W
        pltpu.sync_copy(idx_hbm.at[pl.ds(base, B_PER_W)], idx_v)

        def batch(g):
            r0 = g * KFIRE
            ivec = idx_v[pl.ds(r0, KFIRE)]
            copies = []
            for b in range(KFIRE):
                i = ivec[b]
                cp = pltpu.make_async_copy(
                    emb_hbm.at[i], rows_v.at[r0 + b], sem
                )
                cp.start()
                copies.append(cp)
            for cp in copies:
                cp.wait()

        pl.loop(0, B_PER_W // KFIRE)(batch)
        pltpu.sync_copy(rows_v, out_hbm.at[pl.ds(base, B_PER_W)])

    return _sc_gather


BLK = 4096


def _mlp_body(x_ref, e_ref, wfc_ref, bfc_ref, wattr_ref, battr_ref, out_ref):
    h = jnp.dot(x_ref[:], wfc_ref[:D_IN, :], preferred_element_type=jnp.float32)
    h = h + jnp.dot(e_ref[:], wfc_ref[D_IN:, :], preferred_element_type=jnp.float32)
    h = jnp.maximum(h + bfc_ref[:], 0.0)
    out_ref[:] = (
        jnp.dot(h, wattr_ref[:], preferred_element_type=jnp.float32) + battr_ref[:]
    )


def _tc_mlp(x, e, W_fc, b_fc, W_attr, b_attr):
    return pl.pallas_call(
        _mlp_body,
        grid=(B // BLK,),
        in_specs=[
            pl.BlockSpec((BLK, D_IN), lambda i: (i, 0)),
            pl.BlockSpec((BLK, OBJ_EMBED_DIM), lambda i: (i, 0)),
            pl.BlockSpec((D_IN + OBJ_EMBED_DIM, FC_DIM), lambda i: (0, 0)),
            pl.BlockSpec((1, FC_DIM), lambda i: (0, 0)),
            pl.BlockSpec((FC_DIM, NUM_ATTR), lambda i: (0, 0)),
            pl.BlockSpec((1, NUM_ATTR), lambda i: (0, 0)),
        ],
        out_specs=pl.BlockSpec((BLK, NUM_ATTR), lambda i: (i, 0)),
        out_shape=jax.ShapeDtypeStruct((B, NUM_ATTR), jnp.float32),
    )(x, e, W_fc, b_fc, W_attr, b_attr)


def kernel(x, obj_labels, emb, W_fc, b_fc, W_attr, b_attr):
    e = _get_sc_gather()(emb, obj_labels)
    return _tc_mlp(
        x,
        e,
        W_fc,
        b_fc.reshape(1, FC_DIM),
        W_attr,
        b_attr.reshape(1, NUM_ATTR),
    )


# speedup vs baseline: 1.3229x; 1.0886x over previous
import functools

import jax
import jax.numpy as jnp
from jax import lax
from jax.experimental import pallas as pl
from jax.experimental.pallas import tpu as pltpu
from jax.experimental.pallas import tpu_sc as plsc

B = 16384
D_IN = 256
OBJ_EMBED_DIM = 64
FC_DIM = 256
NUM_ATTR = 400

NC = 2
NS = 16
NW = NC * NS
B_PER_W = B // NW
KFIRE = 16


@functools.cache
def _get_sc_gather():
    mesh = plsc.VectorSubcoreMesh(core_axis_name="c", subcore_axis_name="s")

    @functools.partial(
        pl.kernel,
        mesh=mesh,
        out_type=jax.ShapeDtypeStruct((B, OBJ_EMBED_DIM), jnp.float32),
        scratch_types=[
            pltpu.VMEM((B_PER_W,), jnp.int32),
            pltpu.VMEM((B_PER_W, OBJ_EMBED_DIM), jnp.float32),
            pltpu.SemaphoreType.DMA,
        ],
    )
    def _sc_gather(emb_hbm, idx_hbm, out_hbm, idx_v, rows_v, sem):
        wid = lax.axis_index("s") * NC + lax.axis_index("c")
        base = wid * B_PER_W
        pltpu.sync_copy(idx_hbm.at[pl.ds(base, B_PER_W)], idx_v)

        def fire(r0):
            ivec = idx_v[pl.ds(r0, KFIRE)]
            for b in range(KFIRE):
                pltpu.make_async_copy(
                    emb_hbm.at[ivec[b]], rows_v.at[r0 + b], sem
                ).start()

        def drain(r0):
            for b in range(KFIRE):
                pltpu.make_async_copy(
                    emb_hbm.at[0], rows_v.at[r0 + b], sem
                ).wait()

        nbatch = B_PER_W // KFIRE
        fire(0)

        def body(g):
            r0 = g * KFIRE
            fire(r0 + KFIRE)
            drain(r0)

        pl.loop(0, nbatch - 1)(body)
        drain((nbatch - 1) * KFIRE)
        pltpu.sync_copy(rows_v, out_hbm.at[pl.ds(base, B_PER_W)])

    return _sc_gather


BLK = 4096


def _mlp_body(x_ref, e_ref, wfc_ref, bfc_ref, wattr_ref, battr_ref, out_ref):
    h = jnp.dot(x_ref[:], wfc_ref[:D_IN, :], preferred_element_type=jnp.float32)
    h = h + jnp.dot(e_ref[:], wfc_ref[D_IN:, :], preferred_element_type=jnp.float32)
    h = jnp.maximum(h + bfc_ref[:], 0.0)
    out_ref[:] = (
        jnp.dot(h, wattr_ref[:], preferred_element_type=jnp.float32) + battr_ref[:]
    )


def _tc_mlp(x, e, W_fc, b_fc, W_attr, b_attr):
    return pl.pallas_call(
        _mlp_body,
        grid=(B // BLK,),
        in_specs=[
            pl.BlockSpec((BLK, D_IN), lambda i: (i, 0)),
            pl.BlockSpec((BLK, OBJ_EMBED_DIM), lambda i: (i, 0)),
            pl.BlockSpec((D_IN + OBJ_EMBED_DIM, FC_DIM), lambda i: (0, 0)),
            pl.BlockSpec((1, FC_DIM), lambda i: (0, 0)),
            pl.BlockSpec((FC_DIM, NUM_ATTR), lambda i: (0, 0)),
            pl.BlockSpec((1, NUM_ATTR), lambda i: (0, 0)),
        ],
        out_specs=pl.BlockSpec((BLK, NUM_ATTR), lambda i: (i, 0)),
        out_shape=jax.ShapeDtypeStruct((B, NUM_ATTR), jnp.float32),
    )(x, e, W_fc, b_fc, W_attr, b_attr)


def kernel(x, obj_labels, emb, W_fc, b_fc, W_attr, b_attr):
    e = _get_sc_gather()(emb, obj_labels)
    return _tc_mlp(
        x,
        e,
        W_fc,
        b_fc.reshape(1, FC_DIM),
        W_attr,
        b_attr.reshape(1, NUM_ATTR),
    )


# speedup vs baseline: 1.3710x; 1.0363x over previous
import functools

import jax
import jax.numpy as jnp
from jax import lax
from jax.experimental import pallas as pl
from jax.experimental.pallas import tpu as pltpu
from jax.experimental.pallas import tpu_sc as plsc

B = 16384
D_IN = 256
OBJ_EMBED_DIM = 64
FC_DIM = 256
NUM_ATTR = 400

NC = 2
NS = 16
NW = NC * NS
B_PER_W = B // NW
KFIRE = 32


@functools.cache
def _get_sc_gather():
    mesh = plsc.VectorSubcoreMesh(core_axis_name="c", subcore_axis_name="s")

    @functools.partial(
        pl.kernel,
        mesh=mesh,
        out_type=jax.ShapeDtypeStruct((B, OBJ_EMBED_DIM), jnp.float32),
        scratch_types=[
            pltpu.VMEM((B_PER_W,), jnp.int32),
            pltpu.VMEM((B_PER_W, OBJ_EMBED_DIM), jnp.float32),
            pltpu.SemaphoreType.DMA,
        ],
    )
    def _sc_gather(emb_hbm, idx_hbm, out_hbm, idx_v, rows_v, sem):
        wid = lax.axis_index("s") * NC + lax.axis_index("c")
        base = wid * B_PER_W
        pltpu.sync_copy(idx_hbm.at[pl.ds(base, B_PER_W)], idx_v)

        def fire(r0):
            ivec = idx_v[pl.ds(r0, KFIRE)]
            for b in range(KFIRE):
                pltpu.make_async_copy(
                    emb_hbm.at[ivec[b]], rows_v.at[r0 + b], sem
                ).start()

        def drain(r0):
            for b in range(KFIRE):
                pltpu.make_async_copy(
                    emb_hbm.at[0], rows_v.at[r0 + b], sem
                ).wait()

        nbatch = B_PER_W // KFIRE
        fire(0)

        def body(g):
            r0 = g * KFIRE
            fire(r0 + KFIRE)
            drain(r0)

        pl.loop(0, nbatch - 1)(body)
        drain((nbatch - 1) * KFIRE)
        pltpu.sync_copy(rows_v, out_hbm.at[pl.ds(base, B_PER_W)])

    return _sc_gather


BLK = 4096


def _mlp_body(x_ref, e_ref, wfc_ref, bfc_ref, wattr_ref, battr_ref, out_ref):
    h = jnp.dot(x_ref[:], wfc_ref[:D_IN, :], preferred_element_type=jnp.float32)
    h = h + jnp.dot(e_ref[:], wfc_ref[D_IN:, :], preferred_element_type=jnp.float32)
    h = jnp.maximum(h + bfc_ref[:], 0.0)
    out_ref[:] = (
        jnp.dot(h, wattr_ref[:], preferred_element_type=jnp.float32) + battr_ref[:]
    )


def _tc_mlp(x, e, W_fc, b_fc, W_attr, b_attr):
    return pl.pallas_call(
        _mlp_body,
        grid=(B // BLK,),
        in_specs=[
            pl.BlockSpec((BLK, D_IN), lambda i: (i, 0)),
            pl.BlockSpec((BLK, OBJ_EMBED_DIM), lambda i: (i, 0)),
            pl.BlockSpec((D_IN + OBJ_EMBED_DIM, FC_DIM), lambda i: (0, 0)),
            pl.BlockSpec((1, FC_DIM), lambda i: (0, 0)),
            pl.BlockSpec((FC_DIM, NUM_ATTR), lambda i: (0, 0)),
            pl.BlockSpec((1, NUM_ATTR), lambda i: (0, 0)),
        ],
        out_specs=pl.BlockSpec((BLK, NUM_ATTR), lambda i: (i, 0)),
        out_shape=jax.ShapeDtypeStruct((B, NUM_ATTR), jnp.float32),
    )(x, e, W_fc, b_fc, W_attr, b_attr)


def kernel(x, obj_labels, emb, W_fc, b_fc, W_attr, b_attr):
    e = _get_sc_gather()(emb, obj_labels)
    return _tc_mlp(
        x,
        e,
        W_fc,
        b_fc.reshape(1, FC_DIM),
        W_attr,
        b_attr.reshape(1, NUM_ATTR),
    )


# speedup vs baseline: 1.3956x; 1.0179x over previous
import functools

import jax
import jax.numpy as jnp
from jax import lax
from jax.experimental import pallas as pl
from jax.experimental.pallas import tpu as pltpu
from jax.experimental.pallas import tpu_sc as plsc

B = 16384
D_IN = 256
OBJ_EMBED_DIM = 64
FC_DIM = 256
NUM_ATTR = 400

NC = 2
NS = 16
NW = NC * NS
B_PER_W = B // NW
KFIRE = 64


@functools.cache
def _get_sc_gather():
    mesh = plsc.VectorSubcoreMesh(core_axis_name="c", subcore_axis_name="s")

    @functools.partial(
        pl.kernel,
        mesh=mesh,
        out_type=jax.ShapeDtypeStruct((B, OBJ_EMBED_DIM), jnp.float32),
        scratch_types=[
            pltpu.VMEM((B_PER_W,), jnp.int32),
            pltpu.VMEM((B_PER_W, OBJ_EMBED_DIM), jnp.float32),
            pltpu.SemaphoreType.DMA,
        ],
    )
    def _sc_gather(emb_hbm, idx_hbm, out_hbm, idx_v, rows_v, sem):
        wid = lax.axis_index("s") * NC + lax.axis_index("c")
        base = wid * B_PER_W
        pltpu.sync_copy(idx_hbm.at[pl.ds(base, B_PER_W)], idx_v)

        def fire(r0):
            ivec = idx_v[pl.ds(r0, KFIRE)]
            for b in range(KFIRE):
                pltpu.make_async_copy(
                    emb_hbm.at[ivec[b]], rows_v.at[r0 + b], sem
                ).start()

        def drain(r0):
            for b in range(KFIRE):
                pltpu.make_async_copy(
                    emb_hbm.at[0], rows_v.at[r0 + b], sem
                ).wait()

        nbatch = B_PER_W // KFIRE
        fire(0)

        def body(g):
            r0 = g * KFIRE
            fire(r0 + KFIRE)
            drain(r0)

        pl.loop(0, nbatch - 1)(body)
        drain((nbatch - 1) * KFIRE)
        pltpu.sync_copy(rows_v, out_hbm.at[pl.ds(base, B_PER_W)])

    return _sc_gather


BLK = 4096


def _mlp_body(x_ref, e_ref, wfc_ref, bfc_ref, wattr_ref, battr_ref, out_ref):
    h = jnp.dot(x_ref[:], wfc_ref[:D_IN, :], preferred_element_type=jnp.float32)
    h = h + jnp.dot(e_ref[:], wfc_ref[D_IN:, :], preferred_element_type=jnp.float32)
    h = jnp.maximum(h + bfc_ref[:], 0.0)
    out_ref[:] = (
        jnp.dot(h, wattr_ref[:], preferred_element_type=jnp.float32) + battr_ref[:]
    )


def _tc_mlp(x, e, W_fc, b_fc, W_attr, b_attr):
    return pl.pallas_call(
        _mlp_body,
        grid=(B // BLK,),
        in_specs=[
            pl.BlockSpec((BLK, D_IN), lambda i: (i, 0)),
            pl.BlockSpec((BLK, OBJ_EMBED_DIM), lambda i: (i, 0)),
            pl.BlockSpec((D_IN + OBJ_EMBED_DIM, FC_DIM), lambda i: (0, 0)),
            pl.BlockSpec((1, FC_DIM), lambda i: (0, 0)),
            pl.BlockSpec((FC_DIM, NUM_ATTR), lambda i: (0, 0)),
            pl.BlockSpec((1, NUM_ATTR), lambda i: (0, 0)),
        ],
        out_specs=pl.BlockSpec((BLK, NUM_ATTR), lambda i: (i, 0)),
        out_shape=jax.ShapeDtypeStruct((B, NUM_ATTR), jnp.float32),
    )(x, e, W_fc, b_fc, W_attr, b_attr)


def kernel(x, obj_labels, emb, W_fc, b_fc, W_attr, b_attr):
    e = _get_sc_gather()(emb, obj_labels)
    return _tc_mlp(
        x,
        e,
        W_fc,
        b_fc.reshape(1, FC_DIM),
        W_attr,
        b_attr.reshape(1, NUM_ATTR),
    )


# speedup vs baseline: 1.3962x; 1.0004x over previous
import functools

import jax
import jax.numpy as jnp
from jax import lax
from jax.experimental import pallas as pl
from jax.experimental.pallas import tpu as pltpu
from jax.experimental.pallas import tpu_sc as plsc

B = 16384
D_IN = 256
OBJ_EMBED_DIM = 64
FC_DIM = 256
NUM_ATTR = 400

NC = 2
NS = 16
NW = NC * NS
B_PER_W = B // NW
KFIRE = 128


@functools.cache
def _get_sc_gather():
    mesh = plsc.VectorSubcoreMesh(core_axis_name="c", subcore_axis_name="s")

    @functools.partial(
        pl.kernel,
        mesh=mesh,
        out_type=jax.ShapeDtypeStruct((B, OBJ_EMBED_DIM), jnp.float32),
        scratch_types=[
            pltpu.VMEM((B_PER_W,), jnp.int32),
            pltpu.VMEM((B_PER_W, OBJ_EMBED_DIM), jnp.float32),
            pltpu.SemaphoreType.DMA,
        ],
    )
    def _sc_gather(emb_hbm, idx_hbm, out_hbm, idx_v, rows_v, sem):
        wid = lax.axis_index("s") * NC + lax.axis_index("c")
        base = wid * B_PER_W
        pltpu.sync_copy(idx_hbm.at[pl.ds(base, B_PER_W)], idx_v)

        def fire(r0):
            ivec = idx_v[pl.ds(r0, KFIRE)]
            for b in range(KFIRE):
                pltpu.make_async_copy(
                    emb_hbm.at[ivec[b]], rows_v.at[r0 + b], sem
                ).start()

        def drain(r0):
            for b in range(KFIRE):
                pltpu.make_async_copy(
                    emb_hbm.at[0], rows_v.at[r0 + b], sem
                ).wait()

        nbatch = B_PER_W // KFIRE
        fire(0)

        def body(g):
            r0 = g * KFIRE
            fire(r0 + KFIRE)
            drain(r0)

        pl.loop(0, nbatch - 1)(body)
        drain((nbatch - 1) * KFIRE)
        pltpu.sync_copy(rows_v, out_hbm.at[pl.ds(base, B_PER_W)])

    return _sc_gather


BLK = 4096


def _mlp_body(x_ref, e_ref, wfc_ref, bfc_ref, wattr_ref, battr_ref, out_ref):
    h = jnp.dot(x_ref[:], wfc_ref[:D_IN, :], preferred_element_type=jnp.float32)
    h = h + jnp.dot(e_ref[:], wfc_ref[D_IN:, :], preferred_element_type=jnp.float32)
    h = jnp.maximum(h + bfc_ref[:], 0.0)
    out_ref[:] = (
        jnp.dot(h, wattr_ref[:], preferred_element_type=jnp.float32) + battr_ref[:]
    )


def _tc_mlp(x, e, W_fc, b_fc, W_attr, b_attr):
    return pl.pallas_call(
        _mlp_body,
        grid=(B // BLK,),
        in_specs=[
            pl.BlockSpec((BLK, D_IN), lambda i: (i, 0)),
            pl.BlockSpec((BLK, OBJ_EMBED_DIM), lambda i: (i, 0)),
            pl.BlockSpec((D_IN + OBJ_EMBED_DIM, FC_DIM), lambda i: (0, 0)),
            pl.BlockSpec((1, FC_DIM), lambda i: (0, 0)),
            pl.BlockSpec((FC_DIM, NUM_ATTR), lambda i: (0, 0)),
            pl.BlockSpec((1, NUM_ATTR), lambda i: (0, 0)),
        ],
        out_specs=pl.BlockSpec((BLK, NUM_ATTR), lambda i: (i, 0)),
        out_shape=jax.ShapeDtypeStruct((B, NUM_ATTR), jnp.float32),
    )(x, e, W_fc, b_fc, W_attr, b_attr)


def kernel(x, obj_labels, emb, W_fc, b_fc, W_attr, b_attr):
    e = _get_sc_gather()(emb, obj_labels)
    return _tc_mlp(
        x,
        e,
        W_fc,
        b_fc.reshape(1, FC_DIM),
        W_attr,
        b_attr.reshape(1, NUM_ATTR),
    )


# speedup vs baseline: 1.4059x; 1.0069x over previous
import functools

import jax
import jax.numpy as jnp
from jax import lax
from jax.experimental import pallas as pl
from jax.experimental.pallas import tpu as pltpu
from jax.experimental.pallas import tpu_sc as plsc

B = 16384
D_IN = 256
OBJ_EMBED_DIM = 64
FC_DIM = 256
NUM_ATTR = 400

NC = 2
NS = 16
NW = NC * NS
B_PER_W = B // NW
KFIRE = 128


@functools.cache
def _get_sc_gather():
    mesh = plsc.VectorSubcoreMesh(core_axis_name="c", subcore_axis_name="s")

    @functools.partial(
        pl.kernel,
        mesh=mesh,
        out_type=jax.ShapeDtypeStruct((B, OBJ_EMBED_DIM), jnp.float32),
        scratch_types=[
            pltpu.VMEM((B_PER_W,), jnp.int32),
            pltpu.VMEM((B_PER_W, OBJ_EMBED_DIM), jnp.float32),
            pltpu.SemaphoreType.DMA,
        ],
        compiler_params=pltpu.CompilerParams(skip_device_barrier=True),
    )
    def _sc_gather(emb_hbm, idx_hbm, out_hbm, idx_v, rows_v, sem):
        wid = lax.axis_index("s") * NC + lax.axis_index("c")
        base = wid * B_PER_W
        pltpu.sync_copy(idx_hbm.at[pl.ds(base, B_PER_W)], idx_v)

        def fire(r0):
            ivec = idx_v[pl.ds(r0, KFIRE)]
            for b in range(KFIRE):
                pltpu.make_async_copy(
                    emb_hbm.at[ivec[b]], rows_v.at[r0 + b], sem
                ).start()

        def drain(r0):
            for b in range(KFIRE):
                pltpu.make_async_copy(
                    emb_hbm.at[0], rows_v.at[r0 + b], sem
                ).wait()

        nbatch = B_PER_W // KFIRE
        fire(0)

        def body(g):
            r0 = g * KFIRE
            fire(r0 + KFIRE)
            drain(r0)

        pl.loop(0, nbatch - 1)(body)
        drain((nbatch - 1) * KFIRE)
        pltpu.sync_copy(rows_v, out_hbm.at[pl.ds(base, B_PER_W)])

    return _sc_gather


BLK = 4096


def _mlp_body(x_ref, e_ref, wfc_ref, bfc_ref, wattr_ref, battr_ref, out_ref):
    h = jnp.dot(x_ref[:], wfc_ref[:D_IN, :], preferred_element_type=jnp.float32)
    h = h + jnp.dot(e_ref[:], wfc_ref[D_IN:, :], preferred_element_type=jnp.float32)
    h = jnp.maximum(h + bfc_ref[:], 0.0)
    out_ref[:] = (
        jnp.dot(h, wattr_ref[:], preferred_element_type=jnp.float32) + battr_ref[:]
    )


def _tc_mlp(x, e, W_fc, b_fc, W_attr, b_attr):
    return pl.pallas_call(
        _mlp_body,
        grid=(B // BLK,),
        in_specs=[
            pl.BlockSpec((BLK, D_IN), lambda i: (i, 0)),
            pl.BlockSpec((BLK, OBJ_EMBED_DIM), lambda i: (i, 0)),
            pl.BlockSpec((D_IN + OBJ_EMBED_DIM, FC_DIM), lambda i: (0, 0)),
            pl.BlockSpec((1, FC_DIM), lambda i: (0, 0)),
            pl.BlockSpec((FC_DIM, NUM_ATTR), lambda i: (0, 0)),
            pl.BlockSpec((1, NUM_ATTR), lambda i: (0, 0)),
        ],
        out_specs=pl.BlockSpec((BLK, NUM_ATTR), lambda i: (i, 0)),
        out_shape=jax.ShapeDtypeStruct((B, NUM_ATTR), jnp.float32),
    )(x, e, W_fc, b_fc, W_attr, b_attr)


def kernel(x, obj_labels, emb, W_fc, b_fc, W_attr, b_attr):
    e = _get_sc_gather()(emb, obj_labels)
    return _tc_mlp(
        x,
        e,
        W_fc,
        b_fc.reshape(1, FC_DIM),
        W_attr,
        b_attr.reshape(1, NUM_ATTR),
    )
